# baseline (speedup 1.0000x reference)
import functools

import jax
import jax.numpy as jnp
from jax import lax
from jax.experimental import pallas as pl
from jax.experimental.pallas import tpu as pltpu
from jax.experimental.pallas import tpu_sc as plsc

_T = 2048
_H = 2048
_DFF = 768
_E = 8
_K = 2
_P = _T * _K
_EPAD = 128
_BT = 256
_NTB = _T // _BT
_BM = 256
_NB = (_P + _E * (_BM - 1)) // _BM + 1
_NBBM = _NB * _BM
_NC = 2
_NS = 16
_NW = _NC * _NS
_TW = _T // _NW


def _router_body(x_ref, gwt_ref, wrep_ref, pos_ref, be_ref,
                 carry_ref, rank_s, sel_s):
    i = pl.program_id(0)
    x = x_ref[...]
    logits = lax.dot_general(
        x, gwt_ref[...], (((1,), (0,)), ((), ())),
        preferred_element_type=jnp.float32)
    col = lax.broadcasted_iota(jnp.int32, (_BT, _EPAD), 1)
    valid = col < _E
    logits = jnp.where(valid, logits, jnp.float32(-1e30))
    m = jnp.max(logits, axis=1, keepdims=True)
    p = jnp.exp(logits - m)
    p = p / jnp.sum(p, axis=1, keepdims=True)
    p = jnp.where(valid, p, -1.0)
    m1 = jnp.max(p, axis=1, keepdims=True)
    i1 = jnp.min(jnp.where(p == m1, col, _EPAD), axis=1, keepdims=True)
    p2 = jnp.where(col == i1, -1.0, p)
    m2 = jnp.max(p2, axis=1, keepdims=True)
    i2 = jnp.min(jnp.where(p2 == m2, col, _EPAD), axis=1, keepdims=True)
    denom = m1 + m2
    sel1 = (col == i1)
    sel2 = (col == i2)
    wrep_ref[...] = jnp.where(col < 16, m1 / denom,
                              jnp.where(col < 32, m2 / denom, 0.0))
    s = sel1.astype(jnp.float32) + sel2.astype(jnp.float32)
    rowi = lax.broadcasted_iota(jnp.int32, (_BT, _BT), 0)
    coli = lax.broadcasted_iota(jnp.int32, (_BT, _BT), 1)
    tri = (coli < rowi).astype(jnp.float32)

    @pl.when(i == 0)
    def _():
        carry_ref[...] = jnp.zeros((1, _EPAD), jnp.float32)

    carry = carry_ref[...]
    tot = lax.dot_general(
        tri, s, (((1,), (0,)), ((), ())),
        preferred_element_type=jnp.float32) + carry
    r1 = jnp.sum(jnp.where(sel1, tot, 0.0), axis=1, keepdims=True)
    r2 = jnp.sum(jnp.where(sel2, tot, 0.0), axis=1, keepdims=True)
    rank_s[pl.ds(i * _BT, _BT), :] = (
        jnp.where(col == 0, r1, 0.0)
        + jnp.where(col == 1, r2, 0.0)).astype(jnp.int32)
    sel_s[pl.ds(i * _BT, _BT), :] = (
        jnp.where(col == 0, i1, 0) + jnp.where(col == 1, i2, 0))
    carry_ref[...] = carry + jnp.sum(s, axis=0, keepdims=True)

    @pl.when(i == _NTB - 1)
    def _():
        cnt = carry_ref[...].astype(jnp.int32)
        padded = ((cnt + _BM - 1) // _BM) * _BM
        lane = lax.broadcasted_iota(jnp.int32, (1, _EPAD), 1)
        ranks = rank_s[...]
        sels = sel_s[...]
        colp = lax.broadcasted_iota(jnp.int32, (_T, _EPAD), 1)
        pos = ranks
        be = jnp.zeros((1, _EPAD), jnp.int32)
        startv = lane * _BM
        for e in range(_E):
            base_e = jnp.sum(jnp.where(lane < e, padded, 0))
            end_e = jnp.sum(jnp.where(lane <= e, padded, 0))
            pos = pos + jnp.where((sels == e) & (colp < _K), base_e, 0)
            be = be + jnp.where(startv >= end_e, 1, 0)
        posf = pos.astype(jnp.float32)
        pa_col = jnp.sum(jnp.where(colp == 0, posf, 0.0), axis=1,
                         keepdims=True)
        pb_col = jnp.sum(jnp.where(colp == 1, posf, 0.0), axis=1,
                         keepdims=True)
        rowt = lax.broadcasted_iota(jnp.int32, (_T, _EPAD), 0)
        rmask = (rowt % _EPAD == colp).astype(jnp.float32)
        li = lax.broadcasted_iota(jnp.int32, (_T // _EPAD, _T), 1)
        lr = lax.broadcasted_iota(jnp.int32, (_T // _EPAD, _T), 0)
        lmask = (li // _EPAD == lr).astype(jnp.float32)
        ma = lax.dot_general(
            lmask, rmask * pa_col, (((1,), (0,)), ((), ())),
            precision=lax.Precision.HIGHEST,
            preferred_element_type=jnp.float32)
        mb = lax.dot_general(
            lmask, rmask * pb_col, (((1,), (0,)), ((), ())),
            precision=lax.Precision.HIGHEST,
            preferred_element_type=jnp.float32)
        pos_ref[...] = jnp.concatenate([ma, mb], axis=0).astype(jnp.int32)
        be_ref[...] = jnp.minimum(be, _E - 1)


def _router(x, gate_w):
    gwt = jnp.zeros((_H, _EPAD), jnp.float32).at[:, :_E].set(gate_w.T)
    return pl.pallas_call(
        _router_body,
        grid=(_NTB,),
        in_specs=[
            pl.BlockSpec((_BT, _H), lambda i: (i, 0)),
            pl.BlockSpec((_H, _EPAD), lambda i: (0, 0)),
        ],
        out_specs=[
            pl.BlockSpec((_BT, _EPAD), lambda i: (i, 0)),
            pl.BlockSpec((2 * _T // _EPAD, _EPAD), lambda i: (0, 0)),
            pl.BlockSpec((1, _EPAD), lambda i: (0, 0)),
        ],
        out_shape=[
            jax.ShapeDtypeStruct((_T, _EPAD), jnp.float32),
            jax.ShapeDtypeStruct((2 * _T // _EPAD, _EPAD), jnp.int32),
            jax.ShapeDtypeStruct((1, _EPAD), jnp.int32),
        ],
        scratch_shapes=[
            pltpu.VMEM((1, _EPAD), jnp.float32),
            pltpu.VMEM((_T, _EPAD), jnp.int32),
            pltpu.VMEM((_T, _EPAD), jnp.int32),
        ],
    )(x, gwt)


@functools.cache
def _make_dispatch():
    mesh = plsc.VectorSubcoreMesh(core_axis_name="c", subcore_axis_name="s")
    nch = 2
    ch = _TW // nch

    @functools.partial(
        pl.kernel,
        out_type=jax.ShapeDtypeStruct((_NBBM, _H), jnp.float32),
        mesh=mesh,
        scratch_types=[
            pltpu.VMEM((nch, ch), jnp.int32),
            pltpu.VMEM((nch, ch), jnp.int32),
            pltpu.VMEM((ch, _H), jnp.float32),
            pltpu.SemaphoreType.DMA,
            pltpu.SemaphoreType.DMA,
        ],
    )
    def dispatch(x_hbm, pos_hbm, out_hbm, ia_v, ib_v, rows_v, sa, sb):
        wid = lax.axis_index("s") * _NC + lax.axis_index("c")
        base = wid * _TW
        for c in range(nch):
            pltpu.sync_copy(pos_hbm.at[pl.ds(base + c * ch, ch)],
                            ia_v.at[c])
            pltpu.sync_copy(pos_hbm.at[pl.ds(_T + base + c * ch, ch)],
                            ib_v.at[c])
        for c in range(nch):
            pltpu.sync_copy(x_hbm.at[pl.ds(base + c * ch, ch)], rows_v)
            da = pltpu.async_copy(rows_v, out_hbm.at[ia_v.at[c]], sa)
            db = pltpu.async_copy(rows_v, out_hbm.at[ib_v.at[c]], sb)
            da.wait()
            db.wait()

    return dispatch


def _dispatch(x, pos):
    return _make_dispatch()(x, pos)


def _ffn_body(be_ref, xg_ref, gp_ref, up_ref, dp_ref, out_ref,
              gw_s, uw_s, dw_s):
    b = pl.program_id(0)
    prev = be_ref[0, jnp.maximum(b - 1, 0)]
    changed = (b == 0) | (be_ref[0, b] != prev)

    @pl.when(changed)
    def _():
        gw_s[...] = gp_ref[0].astype(jnp.bfloat16)
        uw_s[...] = up_ref[0].astype(jnp.bfloat16)
        dw_s[...] = dp_ref[0].astype(jnp.bfloat16)

    xb = xg_ref[...].astype(jnp.bfloat16)
    g = lax.dot_general(
        xb, gw_s[...], (((1,), (1,)), ((), ())),
        preferred_element_type=jnp.float32)
    u = lax.dot_general(
        xb, uw_s[...], (((1,), (1,)), ((), ())),
        preferred_element_type=jnp.float32)
    h = (g * lax.logistic(g) * u).astype(jnp.bfloat16)
    y = lax.dot_general(
        h, dw_s[...], (((1,), (1,)), ((), ())),
        preferred_element_type=jnp.float32)
    out_ref[...] = y


def _ffn(be128, xg, gpw, upw, dpw):
    grid_spec = pltpu.PrefetchScalarGridSpec(
        num_scalar_prefetch=1,
        grid=(_NB,),
        in_specs=[
            pl.BlockSpec((_BM, _H), lambda b, be: (b, 0)),
            pl.BlockSpec((1, _DFF, _H), lambda b, be: (be[0, b], 0, 0)),
            pl.BlockSpec((1, _DFF, _H), lambda b, be: (be[0, b], 0, 0)),
            pl.BlockSpec((1, _H, _DFF), lambda b, be: (be[0, b], 0, 0)),
        ],
        out_specs=pl.BlockSpec((_BM, _H), lambda b, be: (b, 0)),
        scratch_shapes=[
            pltpu.VMEM((_DFF, _H), jnp.bfloat16),
            pltpu.VMEM((_DFF, _H), jnp.bfloat16),
            pltpu.VMEM((_H, _DFF), jnp.bfloat16),
        ],
    )
    return pl.pallas_call(
        _ffn_body,
        grid_spec=grid_spec,
        out_shape=jax.ShapeDtypeStruct((_NBBM, _H), jnp.float32),
    )(be128, xg, gpw, upw, dpw)


@functools.cache
def _make_combine():
    mesh = plsc.VectorSubcoreMesh(core_axis_name="c", subcore_axis_name="s")
    ch = 8
    nch = _TW // ch

    @functools.partial(
        pl.kernel,
        out_type=jax.ShapeDtypeStruct((_T, _H), jnp.float32),
        mesh=mesh,
        scratch_types=[
            pltpu.VMEM((_TW, _EPAD), jnp.float32),
            pltpu.VMEM((_TW,), jnp.int32),
            pltpu.VMEM((_TW,), jnp.int32),
            pltpu.VMEM((ch, _H), jnp.float32),
            pltpu.VMEM((ch, _H), jnp.float32),
            pltpu.VMEM((ch, _H), jnp.float32),
            pltpu.VMEM((ch, _H), jnp.float32),
            pltpu.SemaphoreType.DMA,
            pltpu.SemaphoreType.DMA,
        ],
    )
    def combine(yg_hbm, pos_hbm, wrep_hbm, out_hbm,
                w_v, ia_v, ib_v, a0, b0, a1, b1, s0, s1):
        wid = lax.axis_index("s") * _NC + lax.axis_index("c")
        base = wid * _TW
        pltpu.sync_copy(pos_hbm.at[pl.ds(base, _TW)], ia_v)
        pltpu.sync_copy(pos_hbm.at[pl.ds(_T + base, _TW)], ib_v)
        pltpu.sync_copy(wrep_hbm.at[pl.ds(base, _TW)], w_v)
        abufs = (a0, a1)
        bbufs = (b0, b1)
        sems = (s0, s1)

        def issue(c):
            p = c % 2
            da = pltpu.async_copy(
                yg_hbm.at[ia_v.at[pl.ds(c * ch, ch)]], abufs[p], sems[p])
            db = pltpu.async_copy(
                yg_hbm.at[ib_v.at[pl.ds(c * ch, ch)]], bbufs[p], sems[p])
            return (da, db)

        pend = {0: issue(0)}
        for c in range(nch):
            p = c % 2
            da, db = pend[c]
            da.wait()
            db.wait()
            for r in range(ch):
                wa = w_v[c * ch + r, pl.ds(0, 16)]
                wb = w_v[c * ch + r, pl.ds(16, 16)]

                def _mix(k, _, r=r, wa=wa, wb=wb, p=p):
                    off = k * 128
                    for u in range(8):
                        o = off + u * 16
                        abufs[p][r, pl.ds(o, 16)] = (
                            wa * abufs[p][r, pl.ds(o, 16)]
                            + wb * bbufs[p][r, pl.ds(o, 16)])
                    return 0

                lax.fori_loop(0, _H // 128, _mix, 0)
            if c + 1 < nch:
                pend[c + 1] = issue(c + 1)
            pltpu.sync_copy(abufs[p], out_hbm.at[pl.ds(base + c * ch, ch)])

    return combine


def _combine(yg, pos, wrep):
    return _make_combine()(yg, pos, wrep)


@jax.jit
def kernel(hidden_states, gate_w, gate_proj_w, up_proj_w, down_proj_w):
    B, S, H = hidden_states.shape
    x = hidden_states.reshape(-1, H)
    wrep, pos_t, be128 = _router(x, gate_w)
    pos_flat = pos_t.reshape(-1)
    xg = _dispatch(x, pos_flat)
    yg = _ffn(be128, xg, gate_proj_w, up_proj_w, down_proj_w)
    out = _combine(yg, pos_flat, wrep)
    return out.reshape(B, S, H)


# speedup vs baseline: 1.0397x; 1.0397x over previous
import functools

import jax
import jax.numpy as jnp
from jax import lax
from jax.experimental import pallas as pl
from jax.experimental.pallas import tpu as pltpu
from jax.experimental.pallas import tpu_sc as plsc

_T = 2048
_H = 2048
_DFF = 768
_E = 8
_K = 2
_P = _T * _K
_EPAD = 128
_BT = 256
_NTB = _T // _BT
_BM = 256
_NB = (_P + _E * (_BM - 1)) // _BM + 1
_NBBM = _NB * _BM
_NC = 2
_NS = 16
_NW = _NC * _NS
_TW = _T // _NW


def _router_body(x_ref, gwt_ref, wrep_ref, pos_ref, be_ref,
                 carry_ref, rank_s, sel_s):
    i = pl.program_id(0)
    x = x_ref[...]
    logits = lax.dot_general(
        x, gwt_ref[...], (((1,), (0,)), ((), ())),
        preferred_element_type=jnp.float32)
    col = lax.broadcasted_iota(jnp.int32, (_BT, _EPAD), 1)
    valid = col < _E
    logits = jnp.where(valid, logits, jnp.float32(-1e30))
    m = jnp.max(logits, axis=1, keepdims=True)
    p = jnp.exp(logits - m)
    p = p / jnp.sum(p, axis=1, keepdims=True)
    p = jnp.where(valid, p, -1.0)
    m1 = jnp.max(p, axis=1, keepdims=True)
    i1 = jnp.min(jnp.where(p == m1, col, _EPAD), axis=1, keepdims=True)
    p2 = jnp.where(col == i1, -1.0, p)
    m2 = jnp.max(p2, axis=1, keepdims=True)
    i2 = jnp.min(jnp.where(p2 == m2, col, _EPAD), axis=1, keepdims=True)
    denom = m1 + m2
    sel1 = (col == i1)
    sel2 = (col == i2)
    wrep_ref[...] = jnp.where(col < 16, m1 / denom,
                              jnp.where(col < 32, m2 / denom, 0.0))
    s = sel1.astype(jnp.float32) + sel2.astype(jnp.float32)
    rowi = lax.broadcasted_iota(jnp.int32, (_BT, _BT), 0)
    coli = lax.broadcasted_iota(jnp.int32, (_BT, _BT), 1)
    tri = (coli < rowi).astype(jnp.float32)

    @pl.when(i == 0)
    def _():
        carry_ref[...] = jnp.zeros((1, _EPAD), jnp.float32)

    carry = carry_ref[...]
    tot = lax.dot_general(
        tri, s, (((1,), (0,)), ((), ())),
        preferred_element_type=jnp.float32) + carry
    r1 = jnp.sum(jnp.where(sel1, tot, 0.0), axis=1, keepdims=True)
    r2 = jnp.sum(jnp.where(sel2, tot, 0.0), axis=1, keepdims=True)
    rank_s[pl.ds(i * _BT, _BT), :] = (
        jnp.where(col == 0, r1, 0.0)
        + jnp.where(col == 1, r2, 0.0)).astype(jnp.int32)
    sel_s[pl.ds(i * _BT, _BT), :] = (
        jnp.where(col == 0, i1, 0) + jnp.where(col == 1, i2, 0))
    carry_ref[...] = carry + jnp.sum(s, axis=0, keepdims=True)

    @pl.when(i == _NTB - 1)
    def _():
        cnt = carry_ref[...].astype(jnp.int32)
        padded = ((cnt + _BM - 1) // _BM) * _BM
        lane = lax.broadcasted_iota(jnp.int32, (1, _EPAD), 1)
        ranks = rank_s[...]
        sels = sel_s[...]
        colp = lax.broadcasted_iota(jnp.int32, (_T, _EPAD), 1)
        pos = ranks
        be = jnp.zeros((1, _EPAD), jnp.int32)
        startv = lane * _BM
        for e in range(_E):
            base_e = jnp.sum(jnp.where(lane < e, padded, 0))
            end_e = jnp.sum(jnp.where(lane <= e, padded, 0))
            pos = pos + jnp.where((sels == e) & (colp < _K), base_e, 0)
            be = be + jnp.where(startv >= end_e, 1, 0)
        posf = pos.astype(jnp.float32)
        pa_col = jnp.sum(jnp.where(colp == 0, posf, 0.0), axis=1,
                         keepdims=True)
        pb_col = jnp.sum(jnp.where(colp == 1, posf, 0.0), axis=1,
                         keepdims=True)
        rowt = lax.broadcasted_iota(jnp.int32, (_T, _EPAD), 0)
        rmask = (rowt % _EPAD == colp).astype(jnp.float32)
        li = lax.broadcasted_iota(jnp.int32, (_T // _EPAD, _T), 1)
        lr = lax.broadcasted_iota(jnp.int32, (_T // _EPAD, _T), 0)
        lmask = (li // _EPAD == lr).astype(jnp.float32)
        ma = lax.dot_general(
            lmask, rmask * pa_col, (((1,), (0,)), ((), ())),
            precision=lax.Precision.HIGHEST,
            preferred_element_type=jnp.float32)
        mb = lax.dot_general(
            lmask, rmask * pb_col, (((1,), (0,)), ((), ())),
            precision=lax.Precision.HIGHEST,
            preferred_element_type=jnp.float32)
        pos_ref[...] = jnp.concatenate([ma, mb], axis=0).astype(jnp.int32)
        be_ref[...] = jnp.minimum(be, _E - 1)


def _router(x, gate_w):
    gwt = jnp.zeros((_H, _EPAD), jnp.float32).at[:, :_E].set(gate_w.T)
    return pl.pallas_call(
        _router_body,
        grid=(_NTB,),
        in_specs=[
            pl.BlockSpec((_BT, _H), lambda i: (i, 0)),
            pl.BlockSpec((_H, _EPAD), lambda i: (0, 0)),
        ],
        out_specs=[
            pl.BlockSpec((_BT, _EPAD), lambda i: (i, 0)),
            pl.BlockSpec((2 * _T // _EPAD, _EPAD), lambda i: (0, 0)),
            pl.BlockSpec((1, _EPAD), lambda i: (0, 0)),
        ],
        out_shape=[
            jax.ShapeDtypeStruct((_T, _EPAD), jnp.float32),
            jax.ShapeDtypeStruct((2 * _T // _EPAD, _EPAD), jnp.int32),
            jax.ShapeDtypeStruct((1, _EPAD), jnp.int32),
        ],
        scratch_shapes=[
            pltpu.VMEM((1, _EPAD), jnp.float32),
            pltpu.VMEM((_T, _EPAD), jnp.int32),
            pltpu.VMEM((_T, _EPAD), jnp.int32),
        ],
    )(x, gwt)


@functools.cache
def _make_dispatch():
    mesh = plsc.VectorSubcoreMesh(core_axis_name="c", subcore_axis_name="s")
    nch = 2
    ch = _TW // nch

    @functools.partial(
        pl.kernel,
        out_type=jax.ShapeDtypeStruct((_NBBM, _H), jnp.float32),
        mesh=mesh,
        scratch_types=[
            pltpu.VMEM((nch, ch), jnp.int32),
            pltpu.VMEM((nch, ch), jnp.int32),
            pltpu.VMEM((ch, _H), jnp.float32),
            pltpu.SemaphoreType.DMA,
            pltpu.SemaphoreType.DMA,
        ],
    )
    def dispatch(x_hbm, pos_hbm, out_hbm, ia_v, ib_v, rows_v, sa, sb):
        wid = lax.axis_index("s") * _NC + lax.axis_index("c")
        base = wid * _TW
        for c in range(nch):
            pltpu.sync_copy(pos_hbm.at[pl.ds(base + c * ch, ch)],
                            ia_v.at[c])
            pltpu.sync_copy(pos_hbm.at[pl.ds(_T + base + c * ch, ch)],
                            ib_v.at[c])
        for c in range(nch):
            pltpu.sync_copy(x_hbm.at[pl.ds(base + c * ch, ch)], rows_v)
            da = pltpu.async_copy(rows_v, out_hbm.at[ia_v.at[c]], sa)
            db = pltpu.async_copy(rows_v, out_hbm.at[ib_v.at[c]], sb)
            da.wait()
            db.wait()

    return dispatch


def _dispatch(x, pos):
    return _make_dispatch()(x, pos)


def _ffn_body(be_ref, xg_ref, gp_ref, up_ref, dp_ref, out_ref):
    xb = xg_ref[...].astype(jnp.bfloat16)
    gw = gp_ref[0].astype(jnp.bfloat16)
    uw = up_ref[0].astype(jnp.bfloat16)
    g = lax.dot_general(
        xb, gw, (((1,), (1,)), ((), ())),
        preferred_element_type=jnp.float32)
    u = lax.dot_general(
        xb, uw, (((1,), (1,)), ((), ())),
        preferred_element_type=jnp.float32)
    h = (g * lax.logistic(g) * u).astype(jnp.bfloat16)
    dw = dp_ref[0].astype(jnp.bfloat16)
    y = lax.dot_general(
        h, dw, (((1,), (1,)), ((), ())),
        preferred_element_type=jnp.float32)
    out_ref[...] = y


def _ffn(be128, xg, gpw, upw, dpw):
    grid_spec = pltpu.PrefetchScalarGridSpec(
        num_scalar_prefetch=1,
        grid=(_NB,),
        in_specs=[
            pl.BlockSpec((_BM, _H), lambda b, be: (b, 0)),
            pl.BlockSpec((1, _DFF, _H), lambda b, be: (be[0, b], 0, 0)),
            pl.BlockSpec((1, _DFF, _H), lambda b, be: (be[0, b], 0, 0)),
            pl.BlockSpec((1, _H, _DFF), lambda b, be: (be[0, b], 0, 0)),
        ],
        out_specs=pl.BlockSpec((_BM, _H), lambda b, be: (b, 0)),
    )
    return pl.pallas_call(
        _ffn_body,
        grid_spec=grid_spec,
        out_shape=jax.ShapeDtypeStruct((_NBBM, _H), jnp.float32),
    )(be128, xg, gpw, upw, dpw)


@functools.cache
def _make_combine():
    mesh = plsc.VectorSubcoreMesh(core_axis_name="c", subcore_axis_name="s")
    ch = 8
    nch = _TW // ch

    @functools.partial(
        pl.kernel,
        out_type=jax.ShapeDtypeStruct((_T, _H), jnp.float32),
        mesh=mesh,
        scratch_types=[
            pltpu.VMEM((_TW, _EPAD), jnp.float32),
            pltpu.VMEM((_TW,), jnp.int32),
            pltpu.VMEM((_TW,), jnp.int32),
            pltpu.VMEM((ch, _H), jnp.float32),
            pltpu.VMEM((ch, _H), jnp.float32),
            pltpu.VMEM((ch, _H), jnp.float32),
            pltpu.VMEM((ch, _H), jnp.float32),
            pltpu.SemaphoreType.DMA,
            pltpu.SemaphoreType.DMA,
        ],
    )
    def combine(yg_hbm, pos_hbm, wrep_hbm, out_hbm,
                w_v, ia_v, ib_v, a0, b0, a1, b1, s0, s1):
        wid = lax.axis_index("s") * _NC + lax.axis_index("c")
        base = wid * _TW
        pltpu.sync_copy(pos_hbm.at[pl.ds(base, _TW)], ia_v)
        pltpu.sync_copy(pos_hbm.at[pl.ds(_T + base, _TW)], ib_v)
        pltpu.sync_copy(wrep_hbm.at[pl.ds(base, _TW)], w_v)
        abufs = (a0, a1)
        bbufs = (b0, b1)
        sems = (s0, s1)

        def issue(c):
            p = c % 2
            da = pltpu.async_copy(
                yg_hbm.at[ia_v.at[pl.ds(c * ch, ch)]], abufs[p], sems[p])
            db = pltpu.async_copy(
                yg_hbm.at[ib_v.at[pl.ds(c * ch, ch)]], bbufs[p], sems[p])
            return (da, db)

        pend = {0: issue(0)}
        for c in range(nch):
            p = c % 2
            da, db = pend[c]
            da.wait()
            db.wait()
            for r in range(ch):
                wa = w_v[c * ch + r, pl.ds(0, 16)]
                wb = w_v[c * ch + r, pl.ds(16, 16)]

                def _mix(k, _, r=r, wa=wa, wb=wb, p=p):
                    off = k * 128
                    for u in range(8):
                        o = off + u * 16
                        abufs[p][r, pl.ds(o, 16)] = (
                            wa * abufs[p][r, pl.ds(o, 16)]
                            + wb * bbufs[p][r, pl.ds(o, 16)])
                    return 0

                lax.fori_loop(0, _H // 128, _mix, 0)
            if c + 1 < nch:
                pend[c + 1] = issue(c + 1)
            pltpu.sync_copy(abufs[p], out_hbm.at[pl.ds(base + c * ch, ch)])

    return combine


def _combine(yg, pos, wrep):
    return _make_combine()(yg, pos, wrep)


@jax.jit
def kernel(hidden_states, gate_w, gate_proj_w, up_proj_w, down_proj_w):
    B, S, H = hidden_states.shape
    x = hidden_states.reshape(-1, H)
    wrep, pos_t, be128 = _router(x, gate_w)
    pos_flat = pos_t.reshape(-1)
    xg = _dispatch(x, pos_flat)
    yg = _ffn(be128, xg, gate_proj_w, up_proj_w, down_proj_w)
    out = _combine(yg, pos_flat, wrep)
    return out.reshape(B, S, H)
